# baseline (speedup 1.0000x reference)
import functools

import jax
import jax.numpy as jnp
from jax import lax
from jax.experimental import pallas as pl
from jax.experimental.pallas import tpu as pltpu, tpu_sc as plsc

N_NODES = 10000
D = 128
D_OUT = 64
NC = 2
NS = 16
LANES = 16
CHUNK = 80
ZROWS = 48


def _seg_sum_body(with_counts, *refs):
    if with_counts:
        (feats_hbm, src_hbm, dst_hbm, out_hbm, cnt_hbm,
         acc_sh, cnt_sh, sv0, sv1, sv2, sv3, dv0, dv1, dv2, dv3,
         rows0, rows1, rows2, rows3, ones_v, zcnt_v,
         isem0, isem1, isem2, isem3, gsem0, gsem1, gsem2, gsem3) = refs
    else:
        (feats_hbm, src_hbm, dst_hbm, out_hbm,
         acc_sh, sv0, sv1, sv2, sv3, dv0, dv1, dv2, dv3,
         rows0, rows1, rows2, rows3,
         isem0, isem1, isem2, isem3, gsem0, gsem1, gsem2, gsem3) = refs

    sv = [sv0, sv1, sv2, sv3]
    dv = [dv0, dv1, dv2, dv3]
    isem = [isem0, isem1, isem2, isem3]
    rows = [rows0, rows1, rows2, rows3]
    gsem = [gsem0, gsem1, gsem2, gsem3]

    cid = lax.axis_index("c")
    sid = lax.axis_index("s")
    wid = cid * NS + sid

    n_edges = src_hbm.shape[0]
    epw = n_edges // (NC * NS)
    n_chunks = epw // CHUNK
    base = wid * epw
    rows_per_tile = 624
    tail_start = NS * rows_per_tile
    tail_rows = N_NODES - tail_start

    def i_start(c, k):
        off = base + c * CHUNK
        pltpu.async_copy(src_hbm.at[pl.ds(off, CHUNK)], sv[k], isem[k])
        pltpu.async_copy(dst_hbm.at[pl.ds(off, CHUNK)], dv[k], isem[k])

    def i_wait(k):
        pltpu.make_async_copy(src_hbm.at[pl.ds(0, CHUNK)], sv[k], isem[k]).wait()
        pltpu.make_async_copy(dst_hbm.at[pl.ds(0, CHUNK)], dv[k], isem[k]).wait()

    def g_start(r, k):
        pltpu.async_copy(feats_hbm.at[sv[k]], rows[r], gsem[r])

    def g_wait(r, k):
        pltpu.make_async_copy(feats_hbm.at[sv[k]], rows[r], gsem[r]).wait()

    def scat(r, k):
        pltpu.sync_copy(rows[r], acc_sh.at[dv[k]], add=True)
        if with_counts:
            pltpu.sync_copy(ones_v.at[pl.ds(0, CHUNK)], cnt_sh.at[dv[k]], add=True)

    for k in range(4):
        i_start(k, k)

    zero16 = jnp.zeros((LANES,), jnp.float32)

    def zrow_body(i, c):
        for j in range(D // LANES):
            rows0[i, pl.ds(j * LANES, LANES)] = zero16
        return c
    lax.fori_loop(0, ZROWS, zrow_body, 0)

    if with_counts:
        one16 = jnp.ones((LANES,), jnp.float32)

        def zcnt_body(i, c):
            zcnt_v[pl.ds(i * LANES, LANES)] = zero16
            return c
        lax.fori_loop(0, zcnt_v.shape[0] // LANES, zcnt_body, 0)

        for j in range(ones_v.shape[0] // LANES):
            ones_v[pl.ds(j * LANES, LANES)] = one16

    for k in range(rows_per_tile // ZROWS):
        pltpu.sync_copy(rows0.at[pl.ds(0, ZROWS), :],
                        acc_sh.at[pl.ds(sid * rows_per_tile + k * ZROWS, ZROWS), :])

    @pl.when(sid == 0)
    def _():
        pltpu.sync_copy(rows0.at[pl.ds(0, tail_rows), :],
                        acc_sh.at[pl.ds(tail_start, tail_rows), :])

    if with_counts:
        pltpu.sync_copy(zcnt_v, cnt_sh.at[pl.ds(sid * rows_per_tile, rows_per_tile)])

        @pl.when(sid == 0)
        def _():
            pltpu.sync_copy(zcnt_v.at[pl.ds(0, tail_rows)],
                            cnt_sh.at[pl.ds(tail_start, tail_rows)])

    for k in range(3):
        i_wait(k)
        g_start(k, k)

    plsc.subcore_barrier()

    def pipe_body(j, carry):
        c0 = 4 * j
        for k in range(4):
            c = c0 + k
            nk = (k + 3) % 4

            @pl.when(c + 3 < n_chunks)
            def _():
                i_wait(nk)
                g_start(nk, nk)

            g_wait(k, k)
            scat(k, k)

            @pl.when(c + 4 < n_chunks)
            def _():
                i_start(c + 4, k)
        return carry

    lax.fori_loop(0, (n_chunks - 1) // 4, pipe_body, 0)

    g_wait(0, 0)
    scat(0, 0)

    plsc.subcore_barrier()

    pltpu.sync_copy(acc_sh.at[pl.ds(sid * rows_per_tile, rows_per_tile), :],
                    out_hbm.at[cid, pl.ds(sid * rows_per_tile, rows_per_tile), :])

    @pl.when(sid == 0)
    def _():
        pltpu.sync_copy(acc_sh.at[pl.ds(tail_start, tail_rows), :],
                        out_hbm.at[cid, pl.ds(tail_start, tail_rows), :])

    if with_counts:
        pltpu.sync_copy(cnt_sh.at[pl.ds(sid * rows_per_tile, rows_per_tile)], zcnt_v)
        pltpu.sync_copy(zcnt_v,
                        cnt_hbm.at[pl.ds(cid * N_NODES + sid * rows_per_tile,
                                         rows_per_tile)])

        @pl.when(sid == 0)
        def _():
            pltpu.sync_copy(cnt_sh.at[pl.ds(tail_start, tail_rows)],
                            ones_v.at[pl.ds(0, tail_rows)])
            pltpu.sync_copy(ones_v.at[pl.ds(0, tail_rows)],
                            cnt_hbm.at[pl.ds(cid * N_NODES + tail_start, tail_rows)])


def _make_seg_sum(n, with_counts):
    mesh = plsc.VectorSubcoreMesh(core_axis_name="c", subcore_axis_name="s")
    out_type = [jax.ShapeDtypeStruct((NC, n, D), jnp.float32)]
    idx_bufs = [pltpu.VMEM((CHUNK,), jnp.int32) for _ in range(8)]
    row_bufs = [pltpu.VMEM((CHUNK, D), jnp.float32) for _ in range(4)]
    sems = [pltpu.SemaphoreType.DMA] * 8
    if with_counts:
        out_type.append(jax.ShapeDtypeStruct((NC * n,), jnp.float32))
        scratch = ([pltpu.VMEM_SHARED((n, D), jnp.float32),
                    pltpu.VMEM_SHARED((n,), jnp.float32)]
                   + idx_bufs + row_bufs
                   + [pltpu.VMEM((112,), jnp.float32),
                      pltpu.VMEM((624,), jnp.float32)]
                   + sems)
    else:
        scratch = ([pltpu.VMEM_SHARED((n, D), jnp.float32)]
                   + idx_bufs + row_bufs + sems)
    return pl.kernel(
        functools.partial(_seg_sum_body, with_counts),
        out_type=tuple(out_type),
        mesh=mesh,
        scratch_types=scratch,
    )


def _segment_sum_parts(feats, src, dst, with_counts):
    n = feats.shape[0]
    kern = _make_seg_sum(n, with_counts)
    if with_counts:
        agg, cnt_flat = kern(feats, src, dst)
        return agg, cnt_flat.reshape(NC, n)
    return kern(feats, src, dst)[0]


def _combine1_body(agg0, agg1, cnt0, cnt1, x, wl, wr, b, h_ref):
    c = jnp.maximum(cnt0[...] + cnt1[...], 1.0)
    mean = (agg0[...] + agg1[...]) / c
    h = (jnp.dot(mean, wl[...], preferred_element_type=jnp.float32)
         + jnp.dot(x[...], wr[...], preferred_element_type=jnp.float32)
         + b[...])
    h_ref[...] = jnp.where(h >= 0, h, 0.01 * h)


def _combine2_body(agg0, agg1, cnt0, cnt1, x, wl, wr, b, lw, out_ref):
    c = jnp.maximum(cnt0[...] + cnt1[...], 1.0)
    mean = (agg0[...] + agg1[...]) / c
    h = (jnp.dot(mean, wl[...], preferred_element_type=jnp.float32)
         + jnp.dot(x[...], wr[...], preferred_element_type=jnp.float32)
         + b[...])
    h = jnp.where(h >= 0, h, 0.01 * h)
    out_ref[...] = jnp.dot(h, lw[...], preferred_element_type=jnp.float32)


def _row_specs(bn):
    row = pl.BlockSpec((bn, D), lambda i: (i, 0))
    colv = pl.BlockSpec((bn, 1), lambda i: (i, 0))
    wspec = pl.BlockSpec((D, D), lambda i: (0, 0))
    bspec = pl.BlockSpec((1, D), lambda i: (0, 0))
    return row, colv, wspec, bspec


def _combine1(agg, cnt, x, wl, wr, b, bn=1000):
    n = x.shape[0]
    row, colv, wspec, bspec = _row_specs(bn)
    return pl.pallas_call(
        _combine1_body,
        grid=(n // bn,),
        in_specs=[row, row, colv, colv, row, wspec, wspec, bspec],
        out_specs=row,
        out_shape=jax.ShapeDtypeStruct((n, D), jnp.float32),
    )(agg[0], agg[1], cnt[0].reshape(n, 1), cnt[1].reshape(n, 1), x,
      wl, wr, b.reshape(1, D))


def _combine2(agg, cnt, x, wl, wr, b, lw, bn=1000):
    n = x.shape[0]
    row, colv, wspec, bspec = _row_specs(bn)
    return pl.pallas_call(
        _combine2_body,
        grid=(n // bn,),
        in_specs=[row, row, colv, colv, row, wspec, wspec, bspec,
                  pl.BlockSpec((D, D_OUT), lambda i: (0, 0))],
        out_specs=pl.BlockSpec((bn, D_OUT), lambda i: (i, 0)),
        out_shape=jax.ShapeDtypeStruct((n, D_OUT), jnp.float32),
    )(agg[0], agg[1], cnt[0].reshape(n, 1), cnt[1].reshape(n, 1), x,
      wl, wr, b.reshape(1, D), lw)


@jax.jit
def kernel(x, edge_index, W_l1, W_r1, b1, W_l2, W_r2, b2, linear_w):
    src = edge_index[0].astype(jnp.int32)
    dst = edge_index[1].astype(jnp.int32)

    agg1, cnt = _segment_sum_parts(x, src, dst, True)
    h = _combine1(agg1, cnt, x, W_l1, W_r1, b1)
    agg2 = _segment_sum_parts(h, src, dst, False)
    return _combine2(agg2, cnt, h, W_l2, W_r2, b2, linear_w)


# speedup vs baseline: 1.1157x; 1.1157x over previous
import functools

import jax
import jax.numpy as jnp
from jax import lax
from jax.experimental import pallas as pl
from jax.experimental.pallas import tpu as pltpu, tpu_sc as plsc

N_NODES = 10000
D = 128
D_OUT = 64
NC = 2
NS = 16
LANES = 16
CHUNK = 80
ZROWS = 48


def _seg_sum_body(with_counts, *refs):
    if with_counts:
        (feats_hbm, src_hbm, dst_hbm, out_hbm, cnt_hbm,
         acc_sh, cnt_sh, sv0, sv1, sv2, sv3, dv0, dv1, dv2, dv3,
         rows0, rows1, ones_v, zcnt_v,
         isem0, isem1, isem2, isem3, gsem0, gsem1) = refs
    else:
        (feats_hbm, src_hbm, dst_hbm, out_hbm,
         acc_sh, sv0, sv1, sv2, sv3, dv0, dv1, dv2, dv3,
         rows0, rows1,
         isem0, isem1, isem2, isem3, gsem0, gsem1) = refs

    sv = [sv0, sv1, sv2, sv3]
    dv = [dv0, dv1, dv2, dv3]
    isem = [isem0, isem1, isem2, isem3]
    rows = [rows0, rows1]
    gsem = [gsem0, gsem1]

    cid = lax.axis_index("c")
    sid = lax.axis_index("s")
    wid = cid * NS + sid

    n_edges = src_hbm.shape[0]
    epw = n_edges // (NC * NS)
    n_chunks = epw // CHUNK
    base = wid * epw
    rows_per_tile = 624
    tail_start = NS * rows_per_tile
    tail_rows = N_NODES - tail_start

    def i_start(c, k):
        off = base + c * CHUNK
        pltpu.async_copy(src_hbm.at[pl.ds(off, CHUNK)], sv[k], isem[k])
        pltpu.async_copy(dst_hbm.at[pl.ds(off, CHUNK)], dv[k], isem[k])

    def i_wait(k):
        pltpu.make_async_copy(src_hbm.at[pl.ds(0, CHUNK)], sv[k], isem[k]).wait()
        pltpu.make_async_copy(dst_hbm.at[pl.ds(0, CHUNK)], dv[k], isem[k]).wait()

    def g_start(r, k):
        pltpu.async_copy(feats_hbm.at[sv[k]], rows[r], gsem[r])

    def g_wait(r, k):
        pltpu.make_async_copy(feats_hbm.at[sv[k]], rows[r], gsem[r]).wait()

    def scat(r, k):
        pltpu.sync_copy(rows[r], acc_sh.at[dv[k]], add=True)
        if with_counts:
            pltpu.sync_copy(ones_v.at[pl.ds(0, CHUNK)], cnt_sh.at[dv[k]], add=True)

    for k in range(4):
        i_start(k, k)

    zero16 = jnp.zeros((LANES,), jnp.float32)

    def zrow_body(i, c):
        for j in range(D // LANES):
            rows0[i, pl.ds(j * LANES, LANES)] = zero16
        return c
    lax.fori_loop(0, ZROWS, zrow_body, 0)

    if with_counts:
        one16 = jnp.ones((LANES,), jnp.float32)

        def zcnt_body(i, c):
            zcnt_v[pl.ds(i * LANES, LANES)] = zero16
            return c
        lax.fori_loop(0, zcnt_v.shape[0] // LANES, zcnt_body, 0)

        for j in range(ones_v.shape[0] // LANES):
            ones_v[pl.ds(j * LANES, LANES)] = one16

    for k in range(rows_per_tile // ZROWS):
        pltpu.sync_copy(rows0.at[pl.ds(0, ZROWS), :],
                        acc_sh.at[pl.ds(sid * rows_per_tile + k * ZROWS, ZROWS), :])

    @pl.when(sid == 0)
    def _():
        pltpu.sync_copy(rows0.at[pl.ds(0, tail_rows), :],
                        acc_sh.at[pl.ds(tail_start, tail_rows), :])

    if with_counts:
        pltpu.sync_copy(zcnt_v, cnt_sh.at[pl.ds(sid * rows_per_tile, rows_per_tile)])

        @pl.when(sid == 0)
        def _():
            pltpu.sync_copy(zcnt_v.at[pl.ds(0, tail_rows)],
                            cnt_sh.at[pl.ds(tail_start, tail_rows)])

    for k in range(2):
        i_wait(k)
        g_start(k, k)

    plsc.subcore_barrier()

    def pipe_body(j, carry):
        c0 = 4 * j
        for k in range(4):
            c = c0 + k
            nk = (k + 2) % 4

            g_wait(k % 2, k)
            scat(k % 2, k)

            @pl.when(c + 2 < n_chunks)
            def _():
                i_wait(nk)
                g_start(k % 2, nk)

            @pl.when(c + 4 < n_chunks)
            def _():
                i_start(c + 4, k)
        return carry

    lax.fori_loop(0, (n_chunks - 1) // 4, pipe_body, 0)

    g_wait(0, 0)
    scat(0, 0)

    plsc.subcore_barrier()

    pltpu.sync_copy(acc_sh.at[pl.ds(sid * rows_per_tile, rows_per_tile), :],
                    out_hbm.at[cid, pl.ds(sid * rows_per_tile, rows_per_tile), :])

    @pl.when(sid == 0)
    def _():
        pltpu.sync_copy(acc_sh.at[pl.ds(tail_start, tail_rows), :],
                        out_hbm.at[cid, pl.ds(tail_start, tail_rows), :])

    if with_counts:
        pltpu.sync_copy(cnt_sh.at[pl.ds(sid * rows_per_tile, rows_per_tile)], zcnt_v)
        pltpu.sync_copy(zcnt_v,
                        cnt_hbm.at[pl.ds(cid * N_NODES + sid * rows_per_tile,
                                         rows_per_tile)])

        @pl.when(sid == 0)
        def _():
            pltpu.sync_copy(cnt_sh.at[pl.ds(tail_start, tail_rows)],
                            ones_v.at[pl.ds(0, tail_rows)])
            pltpu.sync_copy(ones_v.at[pl.ds(0, tail_rows)],
                            cnt_hbm.at[pl.ds(cid * N_NODES + tail_start, tail_rows)])


def _make_seg_sum(n, with_counts):
    mesh = plsc.VectorSubcoreMesh(core_axis_name="c", subcore_axis_name="s")
    out_type = [jax.ShapeDtypeStruct((NC, n, D), jnp.float32)]
    idx_bufs = [pltpu.VMEM((CHUNK,), jnp.int32) for _ in range(8)]
    row_bufs = [pltpu.VMEM((CHUNK, D), jnp.float32) for _ in range(2)]
    sems = [pltpu.SemaphoreType.DMA] * 6
    if with_counts:
        out_type.append(jax.ShapeDtypeStruct((NC * n,), jnp.float32))
        scratch = ([pltpu.VMEM_SHARED((n, D), jnp.float32),
                    pltpu.VMEM_SHARED((n,), jnp.float32)]
                   + idx_bufs + row_bufs
                   + [pltpu.VMEM((112,), jnp.float32),
                      pltpu.VMEM((624,), jnp.float32)]
                   + sems)
    else:
        scratch = ([pltpu.VMEM_SHARED((n, D), jnp.float32)]
                   + idx_bufs + row_bufs + sems)
    return pl.kernel(
        functools.partial(_seg_sum_body, with_counts),
        out_type=tuple(out_type),
        mesh=mesh,
        scratch_types=scratch,
    )


def _segment_sum_parts(feats, src, dst, with_counts):
    n = feats.shape[0]
    kern = _make_seg_sum(n, with_counts)
    if with_counts:
        agg, cnt_flat = kern(feats, src, dst)
        return agg, cnt_flat.reshape(NC, n)
    return kern(feats, src, dst)[0]


def _combine1_body(agg0, agg1, cnt0, cnt1, x, wl, wr, b, h_ref):
    c = jnp.maximum(cnt0[...] + cnt1[...], 1.0)
    mean = (agg0[...] + agg1[...]) / c
    h = (jnp.dot(mean, wl[...], preferred_element_type=jnp.float32)
         + jnp.dot(x[...], wr[...], preferred_element_type=jnp.float32)
         + b[...])
    h_ref[...] = jnp.where(h >= 0, h, 0.01 * h)


def _combine2_body(agg0, agg1, cnt0, cnt1, x, wl, wr, b, lw, out_ref):
    c = jnp.maximum(cnt0[...] + cnt1[...], 1.0)
    mean = (agg0[...] + agg1[...]) / c
    h = (jnp.dot(mean, wl[...], preferred_element_type=jnp.float32)
         + jnp.dot(x[...], wr[...], preferred_element_type=jnp.float32)
         + b[...])
    h = jnp.where(h >= 0, h, 0.01 * h)
    out_ref[...] = jnp.dot(h, lw[...], preferred_element_type=jnp.float32)


def _row_specs(bn):
    row = pl.BlockSpec((bn, D), lambda i: (i, 0))
    colv = pl.BlockSpec((bn, 1), lambda i: (i, 0))
    wspec = pl.BlockSpec((D, D), lambda i: (0, 0))
    bspec = pl.BlockSpec((1, D), lambda i: (0, 0))
    return row, colv, wspec, bspec


def _combine1(agg, cnt, x, wl, wr, b, bn=1000):
    n = x.shape[0]
    row, colv, wspec, bspec = _row_specs(bn)
    return pl.pallas_call(
        _combine1_body,
        grid=(n // bn,),
        in_specs=[row, row, colv, colv, row, wspec, wspec, bspec],
        out_specs=row,
        out_shape=jax.ShapeDtypeStruct((n, D), jnp.float32),
    )(agg[0], agg[1], cnt[0].reshape(n, 1), cnt[1].reshape(n, 1), x,
      wl, wr, b.reshape(1, D))


def _combine2(agg, cnt, x, wl, wr, b, lw, bn=1000):
    n = x.shape[0]
    row, colv, wspec, bspec = _row_specs(bn)
    return pl.pallas_call(
        _combine2_body,
        grid=(n // bn,),
        in_specs=[row, row, colv, colv, row, wspec, wspec, bspec,
                  pl.BlockSpec((D, D_OUT), lambda i: (0, 0))],
        out_specs=pl.BlockSpec((bn, D_OUT), lambda i: (i, 0)),
        out_shape=jax.ShapeDtypeStruct((n, D_OUT), jnp.float32),
    )(agg[0], agg[1], cnt[0].reshape(n, 1), cnt[1].reshape(n, 1), x,
      wl, wr, b.reshape(1, D), lw)


@jax.jit
def kernel(x, edge_index, W_l1, W_r1, b1, W_l2, W_r2, b2, linear_w):
    src = edge_index[0].astype(jnp.int32)
    dst = edge_index[1].astype(jnp.int32)

    agg1, cnt = _segment_sum_parts(x, src, dst, True)
    h = _combine1(agg1, cnt, x, W_l1, W_r1, b1)
    agg2 = _segment_sum_parts(h, src, dst, False)
    return _combine2(agg2, cnt, h, W_l2, W_r2, b2, linear_w)
